# baseline (speedup 1.0000x reference)
import jax
import jax.numpy as jnp
from jax.experimental import pallas as pl
from jax.experimental.pallas import tpu as pltpu

ROWS, COLS = 8192, 4096
SHIFT = 1024
KEEP = ROWS - SHIFT

N_CHUNKS = 8
CHUNK = KEEP // N_CHUNKS


def _body(buf_ref, x_ref, out_ref, y_ref, sems, xsem, ysem):
    copies = []
    for i in range(N_CHUNKS):
        copies.append(pltpu.make_async_copy(
            buf_ref.at[pl.ds(SHIFT + i * CHUNK, CHUNK)],
            out_ref.at[pl.ds(i * CHUNK, CHUNK)],
            sems.at[i]))
    copies.append(pltpu.make_async_copy(
        x_ref, out_ref.at[pl.ds(KEEP, SHIFT)], xsem))
    copies.append(pltpu.make_async_copy(
        buf_ref.at[pl.ds(0, SHIFT)], y_ref, ysem))
    for c in copies:
        c.start()
    for c in copies:
        c.wait()


def kernel(buffer, x):
    out_buf, y = pl.pallas_call(
        _body,
        in_specs=[
            pl.BlockSpec(memory_space=pl.ANY),
            pl.BlockSpec(memory_space=pl.ANY),
        ],
        out_specs=[
            pl.BlockSpec(memory_space=pl.ANY),
            pl.BlockSpec(memory_space=pl.ANY),
        ],
        out_shape=[
            jax.ShapeDtypeStruct((ROWS, COLS), jnp.float32),
            jax.ShapeDtypeStruct((SHIFT, COLS), jnp.float32),
        ],
        scratch_shapes=[
            pltpu.SemaphoreType.DMA((N_CHUNKS,)),
            pltpu.SemaphoreType.DMA,
            pltpu.SemaphoreType.DMA,
        ],
    )(buffer, x)
    return (out_buf, y)


# speedup vs baseline: 8.6423x; 8.6423x over previous
import functools

import jax
import jax.numpy as jnp
from jax import lax
from jax.experimental import pallas as pl
from jax.experimental.pallas import tpu as pltpu
from jax.experimental.pallas import tpu_sc as plsc

ROWS, COLS = 8192, 4096
SHIFT = 1024
KEEP = ROWS - SHIFT
BLK = 256
GRID = ROWS // BLK
KEEP_BLKS = KEEP // BLK
SHIFT_BLKS = SHIFT // BLK

NC, NS = 2, 16
NW = NC * NS
Y_ROWS_PER_W = SHIFT // NW


def _tc_body(shift_src, x_src, out_ref):
    i = pl.program_id(0)

    @pl.when(i < KEEP_BLKS)
    def _():
        out_ref[...] = shift_src[...]

    @pl.when(i >= KEEP_BLKS)
    def _():
        out_ref[...] = x_src[...]


def _sc_y_body(buf_hbm, y_hbm):
    wid = lax.axis_index("s") * NC + lax.axis_index("c")
    base = wid * Y_ROWS_PER_W
    pltpu.sync_copy(buf_hbm.at[pl.ds(base, Y_ROWS_PER_W)],
                    y_hbm.at[pl.ds(base, Y_ROWS_PER_W)])


_sc_y = functools.partial(
    pl.kernel,
    out_type=jax.ShapeDtypeStruct((SHIFT, COLS), jnp.float32),
    mesh=plsc.VectorSubcoreMesh(
        core_axis_name="c", subcore_axis_name="s",
        num_cores=NC, num_subcores=NS),
)(_sc_y_body)


def kernel(buffer, x):
    y = _sc_y(buffer)
    out_buf = pl.pallas_call(
        _tc_body,
        grid=(GRID,),
        in_specs=[
            pl.BlockSpec((BLK, COLS),
                         lambda i: (jnp.minimum(i + SHIFT_BLKS, GRID - 1), 0)),
            pl.BlockSpec((BLK, COLS),
                         lambda i: (jnp.clip(i - KEEP_BLKS, 0, SHIFT_BLKS - 1), 0)),
        ],
        out_specs=pl.BlockSpec((BLK, COLS), lambda i: (i, 0)),
        out_shape=jax.ShapeDtypeStruct((ROWS, COLS), jnp.float32),
    )(buffer, x)
    return (out_buf, y)


# speedup vs baseline: 40.5315x; 4.6899x over previous
import functools

import jax
import jax.numpy as jnp
from jax import lax
from jax.experimental import pallas as pl
from jax.experimental.pallas import tpu as pltpu
from jax.experimental.pallas import tpu_sc as plsc

ROWS, COLS = 8192, 4096
SHIFT = 1024
KEEP = ROWS - SHIFT
BLK = 256
GRID = ROWS // BLK
KEEP_BLKS = KEEP // BLK
SHIFT_BLKS = SHIFT // BLK

NC, NS = 2, 16
NW = NC * NS
Y_ROWS_PER_W = SHIFT // NW


def _tc_body(shift_src, x_src, out_ref):
    i = pl.program_id(0)

    @pl.when(i < KEEP_BLKS)
    def _():
        out_ref[...] = shift_src[...]

    @pl.when(i >= KEEP_BLKS)
    def _():
        out_ref[...] = x_src[...]


SC_CH = 8
SC_NCH = Y_ROWS_PER_W // SC_CH


def _sc_y_body(buf_hbm, y_hbm, v0, v1, si0, si1, so0, so1):
    wid = lax.axis_index("s") * NC + lax.axis_index("c")
    base = wid * Y_ROWS_PER_W
    vbufs = (v0, v1)
    isems = (si0, si1)
    osems = (so0, so1)

    ins = [None] * SC_NCH
    outs = [None] * SC_NCH
    for k in range(SC_NCH):
        b = k % 2
        if k >= 2:
            outs[k - 2].wait()
        ins[k] = pltpu.async_copy(
            buf_hbm.at[pl.ds(base + k * SC_CH, SC_CH)], vbufs[b], isems[b])
        if k >= 1:
            ins[k - 1].wait()
            outs[k - 1] = pltpu.async_copy(
                vbufs[(k - 1) % 2],
                y_hbm.at[pl.ds(base + (k - 1) * SC_CH, SC_CH)],
                osems[(k - 1) % 2])
    ins[SC_NCH - 1].wait()
    outs[SC_NCH - 1] = pltpu.async_copy(
        vbufs[(SC_NCH - 1) % 2],
        y_hbm.at[pl.ds(base + (SC_NCH - 1) * SC_CH, SC_CH)],
        osems[(SC_NCH - 1) % 2])
    outs[SC_NCH - 2].wait()
    outs[SC_NCH - 1].wait()


_sc_y = functools.partial(
    pl.kernel,
    out_type=jax.ShapeDtypeStruct((SHIFT, COLS), jnp.float32),
    mesh=plsc.VectorSubcoreMesh(
        core_axis_name="c", subcore_axis_name="s",
        num_cores=NC, num_subcores=NS),
    scratch_types=[
        pltpu.VMEM((SC_CH, COLS), jnp.float32),
        pltpu.VMEM((SC_CH, COLS), jnp.float32),
        pltpu.SemaphoreType.DMA,
        pltpu.SemaphoreType.DMA,
        pltpu.SemaphoreType.DMA,
        pltpu.SemaphoreType.DMA,
    ],
)(_sc_y_body)


def kernel(buffer, x):
    y = _sc_y(buffer)
    out_buf = pl.pallas_call(
        _tc_body,
        grid=(GRID,),
        in_specs=[
            pl.BlockSpec((BLK, COLS),
                         lambda i: (jnp.minimum(i + SHIFT_BLKS, GRID - 1), 0)),
            pl.BlockSpec((BLK, COLS),
                         lambda i: (jnp.clip(i - KEEP_BLKS, 0, SHIFT_BLKS - 1), 0)),
        ],
        out_specs=pl.BlockSpec((BLK, COLS), lambda i: (i, 0)),
        out_shape=jax.ShapeDtypeStruct((ROWS, COLS), jnp.float32),
    )(buffer, x)
    return (out_buf, y)


# speedup vs baseline: 44.1058x; 1.0882x over previous
import jax
import jax.numpy as jnp
from jax.experimental import pallas as pl
from jax.experimental.pallas import tpu as pltpu

ROWS, COLS = 8192, 4096
SHIFT = 1024
KEEP = ROWS - SHIFT
BLK = 128
GRID = ROWS // BLK
KEEP_BLKS = KEEP // BLK
SHIFT_BLKS = SHIFT // BLK


def _body(shift_src, y_src, x_src, out_ref, y_ref):
    i = pl.program_id(0)

    @pl.when(i < KEEP_BLKS)
    def _():
        out_ref[...] = shift_src[...]

    @pl.when(i >= KEEP_BLKS)
    def _():
        out_ref[...] = x_src[...]

    @pl.when(i < SHIFT_BLKS)
    def _():
        y_ref[...] = y_src[...]


def kernel(buffer, x):
    out_buf, y = pl.pallas_call(
        _body,
        grid=(GRID,),
        in_specs=[
            pl.BlockSpec((BLK, COLS),
                         lambda i: (jnp.minimum(i + SHIFT_BLKS, GRID - 1), 0)),
            pl.BlockSpec((BLK, COLS),
                         lambda i: (jnp.minimum(i, SHIFT_BLKS - 1), 0)),
            pl.BlockSpec((BLK, COLS),
                         lambda i: (jnp.clip(i - KEEP_BLKS, 0, SHIFT_BLKS - 1), 0)),
        ],
        out_specs=[
            pl.BlockSpec((BLK, COLS), lambda i: (i, 0)),
            pl.BlockSpec((BLK, COLS),
                         lambda i: (jnp.minimum(i, SHIFT_BLKS - 1), 0)),
        ],
        out_shape=[
            jax.ShapeDtypeStruct((ROWS, COLS), jnp.float32),
            jax.ShapeDtypeStruct((SHIFT, COLS), jnp.float32),
        ],
    )(buffer, buffer, x)
    return (out_buf, y)


# speedup vs baseline: 48.7599x; 1.1055x over previous
import jax
import jax.numpy as jnp
from jax.experimental import pallas as pl
from jax.experimental.pallas import tpu as pltpu

ROWS, COLS = 8192, 4096
SHIFT = 1024
KEEP = ROWS - SHIFT
BLK = 512
GRID = ROWS // BLK
KEEP_BLKS = KEEP // BLK
SHIFT_BLKS = SHIFT // BLK
YBLK = SHIFT // GRID


def _body(shift_src, y_src, x_src, out_ref, y_ref):
    i = pl.program_id(0)

    @pl.when(i < KEEP_BLKS)
    def _():
        out_ref[...] = shift_src[...]

    @pl.when(i >= KEEP_BLKS)
    def _():
        out_ref[...] = x_src[...]

    y_ref[...] = y_src[...]


def kernel(buffer, x):
    out_buf, y = pl.pallas_call(
        _body,
        grid=(GRID,),
        in_specs=[
            pl.BlockSpec((BLK, COLS),
                         lambda i: (jnp.minimum(i + SHIFT_BLKS, GRID - 1), 0)),
            pl.BlockSpec((YBLK, COLS), lambda i: (i, 0)),
            pl.BlockSpec((BLK, COLS),
                         lambda i: (jnp.clip(i - KEEP_BLKS, 0, SHIFT_BLKS - 1), 0)),
        ],
        out_specs=[
            pl.BlockSpec((BLK, COLS), lambda i: (i, 0)),
            pl.BlockSpec((YBLK, COLS), lambda i: (i, 0)),
        ],
        out_shape=[
            jax.ShapeDtypeStruct((ROWS, COLS), jnp.float32),
            jax.ShapeDtypeStruct((SHIFT, COLS), jnp.float32),
        ],
    )(buffer, buffer, x)
    return (out_buf, y)
